# baseline (speedup 1.0000x reference)
import functools

import jax
import jax.numpy as jnp
from jax import lax
from jax.experimental import pallas as pl
from jax.experimental.pallas import tpu as pltpu
from jax.experimental.pallas import tpu_sc as plsc

N = 10000
E = 320000
D = 128
DH = D // 2

NC = 2
NS = 16
EPW = E // NS
CHUNK = 125
NCH = EPW // CHUNK
NBUF = 4
NG = NCH // NBUF
RPAD = 10240
RPS = RPAD // NS

_mesh = plsc.VectorSubcoreMesh(core_axis_name="c", subcore_axis_name="s")


def _make_agg(with_counts: bool):
  out_type = [jax.ShapeDtypeStruct((NC, RPAD, DH), jnp.float32)]
  scratch = [
      pltpu.VMEM((NCH, CHUNK), jnp.int32),
      pltpu.VMEM((NCH, CHUNK), jnp.int32),
      [pltpu.VMEM((CHUNK, DH), jnp.float32) for _ in range(NBUF)],
      pltpu.VMEM_SHARED((RPAD, DH), jnp.float32),
      [pltpu.SemaphoreType.DMA for _ in range(NBUF)],
      [pltpu.SemaphoreType.DMA for _ in range(NBUF)],
  ]
  if with_counts:
    out_type.append(jax.ShapeDtypeStruct((NC, RPAD, 16), jnp.float32))
    scratch.append(pltpu.VMEM((CHUNK, 16), jnp.float32))
    scratch.append(pltpu.VMEM_SHARED((RPAD, 16), jnp.float32))
    scratch.append(pltpu.SemaphoreType.DMA)

  @functools.partial(pl.kernel, mesh=_mesh, out_type=out_type,
                     scratch_types=scratch,
                     compiler_params=pltpu.CompilerParams(
                         use_tc_tiling_on_sc=False))
  def agg(*refs):
    if with_counts:
      (feat2, src3, dst3, zrow, zcnt, ones_h,
       agg_out, cnt_out, src_v, dst_v, rows, acc_s, gsem, ssem,
       ones_v, cnt_s, csem) = refs
    else:
      (feat2, src3, dst3, zrow,
       agg_out, src_v, dst_v, rows, acc_s, gsem, ssem) = refs
    cid = lax.axis_index("c")
    sid = lax.axis_index("s")
    row0 = sid * RPS
    pltpu.sync_copy(zrow, acc_s.at[pl.ds(row0, RPS)])
    pltpu.sync_copy(src3.at[sid], src_v)
    pltpu.sync_copy(dst3.at[sid], dst_v)
    if with_counts:
      pltpu.sync_copy(zcnt, cnt_s.at[pl.ds(row0, RPS)])
      pltpu.sync_copy(ones_h, ones_v)
    plsc.subcore_barrier()

    def gather(i, b):
      pltpu.async_copy(feat2.at[cid].at[src_v.at[i]], rows[b], gsem[b])

    def gwait(i, b):
      pltpu.make_async_copy(feat2.at[cid].at[src_v.at[i]], rows[b],
                            gsem[b]).wait()

    def swait(i, b):
      pltpu.make_async_copy(rows[b], acc_s.at[dst_v.at[i]], ssem[b]).wait()

    for b in range(NBUF):
      gather(b, b)

    def scatter_group(i0):
      for b in range(NBUF):
        i = i0 + b
        gwait(i, b)
        pltpu.async_copy(rows[b], acc_s.at[dst_v.at[i]], ssem[b], add=True)
        if with_counts:
          @pl.when(cid == b % 2)
          def _():
            pltpu.async_copy(ones_v, cnt_s.at[dst_v.at[i]], csem, add=True)

    def body(j, carry):
      i0 = NBUF * j
      scatter_group(i0)
      for b in range(NBUF):
        i = i0 + b
        swait(i, b)
        gather(i + NBUF, b)
      return carry

    lax.fori_loop(0, NG - 1, body, 0)
    scatter_group(NCH - NBUF)
    for b in range(NBUF):
      swait(NCH - NBUF + b, b)
    if with_counts:
      def cdrain(_, carry):
        pltpu.make_async_copy(ones_v, cnt_s.at[dst_v.at[0]], csem).wait()
        return carry
      lax.fori_loop(0, NCH // 2, cdrain, 0)
    plsc.subcore_barrier()
    pltpu.sync_copy(acc_s.at[pl.ds(row0, RPS)],
                    agg_out.at[cid, pl.ds(row0, RPS)])
    if with_counts:
      pltpu.sync_copy(cnt_s.at[pl.ds(row0, RPS)],
                      cnt_out.at[cid, pl.ds(row0, RPS)])

  return agg


_agg_counts = _make_agg(True)
_agg_plain = _make_agg(False)

_R = 1000


def _make_dense(relu: bool, split_out: bool):
  def body(agg_ref, cnt_ref, x_ref, wl_ref, b_ref, wr_ref, *o_refs):
    cnt = cnt_ref[0, :, 0:1] + cnt_ref[1, :, 0:1]
    inv = 1.0 / jnp.maximum(cnt, 1.0)
    y = (jnp.dot(agg_ref[0] * inv, wl_ref[0:DH, :],
                 preferred_element_type=jnp.float32)
         + jnp.dot(agg_ref[1] * inv, wl_ref[DH:D, :],
                   preferred_element_type=jnp.float32)
         + b_ref[...]
         + jnp.dot(x_ref[0], wr_ref[0:DH, :],
                   preferred_element_type=jnp.float32)
         + jnp.dot(x_ref[1], wr_ref[DH:D, :],
                   preferred_element_type=jnp.float32))
    if relu:
      y = jnp.maximum(y, 0.0)
    if split_out:
      o_refs[0][0] = y[:, 0:DH]
      o_refs[0][1] = y[:, DH:D]
    else:
      o_refs[0][...] = y

  if split_out:
    out_shape = jax.ShapeDtypeStruct((NC, N, DH), jnp.float32)
    out_specs = pl.BlockSpec((NC, _R, DH), lambda i: (0, i, 0))
  else:
    out_shape = jax.ShapeDtypeStruct((N, D), jnp.float32)
    out_specs = pl.BlockSpec((_R, D), lambda i: (i, 0))

  return pl.pallas_call(
      body,
      grid=(N // _R,),
      in_specs=[
          pl.BlockSpec((NC, _R, DH), lambda i: (0, i, 0)),
          pl.BlockSpec((NC, _R, 16), lambda i: (0, i, 0)),
          pl.BlockSpec((NC, _R, DH), lambda i: (0, i, 0)),
          pl.BlockSpec((D, D), lambda i: (0, 0)),
          pl.BlockSpec((1, D), lambda i: (0, 0)),
          pl.BlockSpec((D, D), lambda i: (0, 0)),
      ],
      out_specs=out_specs,
      out_shape=out_shape,
  )


_dense_relu_split = _make_dense(True, True)
_dense_lin = _make_dense(False, False)


def kernel(x, edge_index, W1_l, b1_l, W1_r, W2_l, b2_l, W2_r):
  src = edge_index[0].astype(jnp.int32).reshape(NS, NCH, CHUNK)
  dst = edge_index[1].astype(jnp.int32).reshape(NS, NCH, CHUNK)
  xs = jnp.moveaxis(x.reshape(N, NC, DH), 1, 0)
  zrow = jnp.zeros((RPS, DH), jnp.float32)
  zcnt = jnp.zeros((RPS, 16), jnp.float32)
  ones = jnp.ones((CHUNK, 16), jnp.float32)
  b1 = b1_l.reshape(1, D)
  b2 = b2_l.reshape(1, D)

  agg1, cnt = _agg_counts(xs, src, dst, zrow, zcnt, ones)
  hs = _dense_relu_split(agg1, cnt, xs, W1_l, b1, W1_r)
  agg2, = _agg_plain(hs, src, dst, zrow)
  return _dense_lin(agg2, cnt, hs, W2_l, b2, W2_r)


# speedup vs baseline: 1.0574x; 1.0574x over previous
import functools

import jax
import jax.numpy as jnp
from jax import lax
from jax.experimental import pallas as pl
from jax.experimental.pallas import tpu as pltpu
from jax.experimental.pallas import tpu_sc as plsc

N = 10000
E = 320000
D = 128
DH = D // 2

NC = 2
NS = 16
EPW = E // NS
CHUNK = 125
NCH = EPW // CHUNK
NBUF = 8
NG = NCH // NBUF
RPAD = 10240
RPS = RPAD // NS

_mesh = plsc.VectorSubcoreMesh(core_axis_name="c", subcore_axis_name="s")


def _make_agg(with_counts: bool):
  out_type = [jax.ShapeDtypeStruct((NC, RPAD, DH), jnp.float32)]
  scratch = [
      pltpu.VMEM((2, NBUF, 2, CHUNK), jnp.int32),
      [pltpu.VMEM((CHUNK, DH), jnp.float32) for _ in range(NBUF)],
      pltpu.VMEM_SHARED((RPAD, DH), jnp.float32),
      [pltpu.SemaphoreType.DMA for _ in range(2)],
      [pltpu.SemaphoreType.DMA for _ in range(NBUF)],
      [pltpu.SemaphoreType.DMA for _ in range(NBUF)],
  ]
  if with_counts:
    out_type.append(jax.ShapeDtypeStruct((NC, RPAD, 16), jnp.float32))
    scratch.append(pltpu.VMEM((CHUNK, 16), jnp.float32))
    scratch.append(pltpu.VMEM_SHARED((RPAD, 16), jnp.float32))
    scratch.append(pltpu.SemaphoreType.DMA)

  @functools.partial(pl.kernel, mesh=_mesh, out_type=out_type,
                     scratch_types=scratch,
                     compiler_params=pltpu.CompilerParams(
                         use_tc_tiling_on_sc=False))
  def agg(*refs):
    if with_counts:
      (feat2, edges4, zrow, zcnt, ones_h,
       agg_out, cnt_out, ev, rows, acc_s, isem, gsem, ssem,
       ones_v, cnt_s, csem) = refs
    else:
      (feat2, edges4, zrow,
       agg_out, ev, rows, acc_s, isem, gsem, ssem) = refs
    cid = lax.axis_index("c")
    sid = lax.axis_index("s")
    row0 = sid * RPS
    pltpu.sync_copy(zrow, acc_s.at[pl.ds(row0, RPS)])
    if with_counts:
      pltpu.sync_copy(zcnt, cnt_s.at[pl.ds(row0, RPS)])
      pltpu.sync_copy(ones_h, ones_v)

    def iload(j, g):
      pltpu.async_copy(edges4.at[sid, j], ev.at[g], isem[g])

    def iwait(j, g):
      pltpu.make_async_copy(edges4.at[sid, j], ev.at[g], isem[g]).wait()

    def gather(i, b, g):
      pltpu.async_copy(feat2.at[cid].at[ev.at[g, b, 0]], rows[b], gsem[b])

    def gwait(i, b, g):
      pltpu.make_async_copy(feat2.at[cid].at[ev.at[g, b, 0]], rows[b],
                            gsem[b]).wait()

    def swait(b, g):
      pltpu.make_async_copy(rows[b], acc_s.at[ev.at[g, b, 1]], ssem[b]).wait()

    plsc.subcore_barrier()
    iload(0, 0)
    iwait(0, 0)
    iload(1, 1)
    for b in range(NBUF):
      gather(b, b, 0)

    def scatter_group(i0, g):
      for b in range(NBUF):
        i = i0 + b
        gwait(i, b, g)
        pltpu.async_copy(rows[b], acc_s.at[ev.at[g, b, 1]], ssem[b], add=True)
        if with_counts:
          @pl.when(cid == b % 2)
          def _():
            pltpu.async_copy(ones_v, cnt_s.at[ev.at[g, b, 1]], csem, add=True)

    def process_group(j, g, gn, do_iload):
      scatter_group(NBUF * j, g)
      for b in range(NBUF):
        swait(b, g)
        if b == 0:
          iwait(j + 1, gn)
        gather(NBUF * (j + 1) + b, b, gn)
      if do_iload:
        iload(j + 2, g)

    def body(k, carry):
      process_group(2 * k, 0, 1, True)
      process_group(2 * k + 1, 1, 0, True)
      return carry

    lax.fori_loop(0, NG // 2 - 1, body, 0)
    process_group(NG - 2, 0, 1, False)
    scatter_group(NCH - NBUF, 1)
    for b in range(NBUF):
      swait(b, 1)
    if with_counts:
      def cdrain(_, carry):
        pltpu.make_async_copy(ones_v, cnt_s.at[ev.at[0, 0, 1]], csem).wait()
        return carry
      lax.fori_loop(0, NCH // 2, cdrain, 0)
    plsc.subcore_barrier()
    pltpu.sync_copy(acc_s.at[pl.ds(row0, RPS)],
                    agg_out.at[cid, pl.ds(row0, RPS)])
    if with_counts:
      pltpu.sync_copy(cnt_s.at[pl.ds(row0, RPS)],
                      cnt_out.at[cid, pl.ds(row0, RPS)])

  return agg


_agg_counts = _make_agg(True)
_agg_plain = _make_agg(False)

_R = 1000


def _make_dense(relu: bool, split_out: bool):
  def body(agg_ref, cnt_ref, x_ref, wl_ref, b_ref, wr_ref, *o_refs):
    cnt = cnt_ref[0, :, 0:1] + cnt_ref[1, :, 0:1]
    inv = 1.0 / jnp.maximum(cnt, 1.0)
    y = (jnp.dot(agg_ref[0] * inv, wl_ref[0:DH, :],
                 preferred_element_type=jnp.float32)
         + jnp.dot(agg_ref[1] * inv, wl_ref[DH:D, :],
                   preferred_element_type=jnp.float32)
         + b_ref[...]
         + jnp.dot(x_ref[0], wr_ref[0:DH, :],
                   preferred_element_type=jnp.float32)
         + jnp.dot(x_ref[1], wr_ref[DH:D, :],
                   preferred_element_type=jnp.float32))
    if relu:
      y = jnp.maximum(y, 0.0)
    if split_out:
      o_refs[0][0] = y[:, 0:DH]
      o_refs[0][1] = y[:, DH:D]
    else:
      o_refs[0][...] = y

  if split_out:
    out_shape = jax.ShapeDtypeStruct((NC, N, DH), jnp.float32)
    out_specs = pl.BlockSpec((NC, _R, DH), lambda i: (0, i, 0))
  else:
    out_shape = jax.ShapeDtypeStruct((N, D), jnp.float32)
    out_specs = pl.BlockSpec((_R, D), lambda i: (i, 0))

  return pl.pallas_call(
      body,
      grid=(N // _R,),
      in_specs=[
          pl.BlockSpec((NC, _R, DH), lambda i: (0, i, 0)),
          pl.BlockSpec((NC, _R, 16), lambda i: (0, i, 0)),
          pl.BlockSpec((NC, _R, DH), lambda i: (0, i, 0)),
          pl.BlockSpec((D, D), lambda i: (0, 0)),
          pl.BlockSpec((1, D), lambda i: (0, 0)),
          pl.BlockSpec((D, D), lambda i: (0, 0)),
      ],
      out_specs=out_specs,
      out_shape=out_shape,
  )


_dense_relu_split = _make_dense(True, True)
_dense_lin = _make_dense(False, False)


def kernel(x, edge_index, W1_l, b1_l, W1_r, W2_l, b2_l, W2_r):
  ei = edge_index.astype(jnp.int32).reshape(2, NS, NG, NBUF, CHUNK)
  edges4 = jnp.moveaxis(ei, 0, 3)
  xs = jnp.moveaxis(x.reshape(N, NC, DH), 1, 0)
  zrow = jnp.zeros((RPS, DH), jnp.float32)
  zcnt = jnp.zeros((RPS, 16), jnp.float32)
  ones = jnp.ones((CHUNK, 16), jnp.float32)
  b1 = b1_l.reshape(1, D)
  b2 = b2_l.reshape(1, D)

  agg1, cnt = _agg_counts(xs, edges4, zrow, zcnt, ones)
  hs = _dense_relu_split(agg1, cnt, xs, W1_l, b1, W1_r)
  agg2, = _agg_plain(hs, edges4, zrow)
  return _dense_lin(agg2, cnt, hs, W2_l, b2, W2_r)


# speedup vs baseline: 1.0657x; 1.0079x over previous
import functools

import jax
import jax.numpy as jnp
from jax import lax
from jax.experimental import pallas as pl
from jax.experimental.pallas import tpu as pltpu
from jax.experimental.pallas import tpu_sc as plsc

N = 10000
E = 320000
D = 128
DH = D // 2

NC = 2
NS = 16
EPW = E // NS
CHUNK = 125
NCH = EPW // CHUNK
NBUF = 8
NG = NCH // NBUF
RPAD = 10240
RPS = RPAD // NS

_mesh = plsc.VectorSubcoreMesh(core_axis_name="c", subcore_axis_name="s")


def _make_agg(with_counts: bool):
  out_type = [jax.ShapeDtypeStruct((NC, RPAD, DH), jnp.float32)]
  scratch = [
      pltpu.VMEM((2, 2, NBUF, CHUNK), jnp.int32),
      [pltpu.VMEM((CHUNK, DH), jnp.float32) for _ in range(NBUF)],
      pltpu.VMEM_SHARED((RPAD, DH), jnp.float32),
      [pltpu.SemaphoreType.DMA for _ in range(2)],
      [pltpu.SemaphoreType.DMA for _ in range(NBUF)],
      [pltpu.SemaphoreType.DMA for _ in range(NBUF)],
  ]
  if with_counts:
    out_type.append(jax.ShapeDtypeStruct((NC, RPAD, 16), jnp.float32))
    scratch.append(pltpu.VMEM((CHUNK, 16), jnp.float32))
    scratch.append(pltpu.VMEM_SHARED((RPAD, 16), jnp.float32))
    scratch.append(pltpu.SemaphoreType.DMA)

  @functools.partial(pl.kernel, mesh=_mesh, out_type=out_type,
                     scratch_types=scratch,
                     compiler_params=pltpu.CompilerParams(
                         use_tc_tiling_on_sc=False))
  def agg(*refs):
    if with_counts:
      (feat2, edges4, zrow, zcnt, ones_h,
       agg_out, cnt_out, ev, rows, acc_s, isem, gsem, ssem,
       ones_v, cnt_s, csem) = refs
    else:
      (feat2, edges4, zrow,
       agg_out, ev, rows, acc_s, isem, gsem, ssem) = refs
    cid = lax.axis_index("c")
    sid = lax.axis_index("s")
    row0 = sid * RPS
    pltpu.sync_copy(zrow, acc_s.at[pl.ds(row0, RPS)])
    if with_counts:
      pltpu.sync_copy(zcnt, cnt_s.at[pl.ds(row0, RPS)])
      pltpu.sync_copy(ones_h, ones_v)

    def iload(j, g):
      pltpu.async_copy(edges4.at[0, sid, j], ev.at[g, 0], isem[g])
      pltpu.async_copy(edges4.at[1, sid, j], ev.at[g, 1], isem[g])

    def iwait(j, g):
      pltpu.make_async_copy(edges4.at[0, sid, j], ev.at[g, 0], isem[g]).wait()
      pltpu.make_async_copy(edges4.at[1, sid, j], ev.at[g, 1], isem[g]).wait()

    def gather(i, b, g):
      pltpu.async_copy(feat2.at[cid].at[ev.at[g, 0, b]], rows[b], gsem[b])

    def gwait(i, b, g):
      pltpu.make_async_copy(feat2.at[cid].at[ev.at[g, 0, b]], rows[b],
                            gsem[b]).wait()

    def swait(b, g):
      pltpu.make_async_copy(rows[b], acc_s.at[ev.at[g, 1, b]], ssem[b]).wait()

    plsc.subcore_barrier()
    iload(0, 0)
    iwait(0, 0)
    iload(1, 1)
    for b in range(NBUF):
      gather(b, b, 0)

    def scatter_group(i0, g):
      for b in range(NBUF):
        i = i0 + b
        gwait(i, b, g)
        pltpu.async_copy(rows[b], acc_s.at[ev.at[g, 1, b]], ssem[b], add=True)
        if with_counts:
          @pl.when(cid == b % 2)
          def _():
            pltpu.async_copy(ones_v, cnt_s.at[ev.at[g, 1, b]], csem, add=True)

    def process_group(j, g, gn, do_iload):
      scatter_group(NBUF * j, g)
      for b in range(NBUF):
        swait(b, g)
        if b == 0:
          iwait(j + 1, gn)
        gather(NBUF * (j + 1) + b, b, gn)
      if do_iload:
        iload(j + 2, g)

    def body(k, carry):
      process_group(2 * k, 0, 1, True)
      process_group(2 * k + 1, 1, 0, True)
      return carry

    lax.fori_loop(0, NG // 2 - 1, body, 0)
    process_group(NG - 2, 0, 1, False)
    scatter_group(NCH - NBUF, 1)
    for b in range(NBUF):
      swait(b, 1)
    if with_counts:
      def cdrain(_, carry):
        pltpu.make_async_copy(ones_v, cnt_s.at[ev.at[0, 1, 0]], csem).wait()
        return carry
      lax.fori_loop(0, NCH // 2, cdrain, 0)
    plsc.subcore_barrier()
    pltpu.sync_copy(acc_s.at[pl.ds(row0, RPS)],
                    agg_out.at[cid, pl.ds(row0, RPS)])
    if with_counts:
      pltpu.sync_copy(cnt_s.at[pl.ds(row0, RPS)],
                      cnt_out.at[cid, pl.ds(row0, RPS)])

  return agg


_agg_counts = _make_agg(True)
_agg_plain = _make_agg(False)

_R = 1000


def _make_dense(relu: bool, split_out: bool):
  def body(agg_ref, cnt_ref, x_ref, wl_ref, b_ref, wr_ref, *o_refs):
    cnt = cnt_ref[0, :, 0:1] + cnt_ref[1, :, 0:1]
    inv = 1.0 / jnp.maximum(cnt, 1.0)
    y = (jnp.dot(agg_ref[0] * inv, wl_ref[0:DH, :],
                 preferred_element_type=jnp.float32)
         + jnp.dot(agg_ref[1] * inv, wl_ref[DH:D, :],
                   preferred_element_type=jnp.float32)
         + b_ref[...]
         + jnp.dot(x_ref[0], wr_ref[0:DH, :],
                   preferred_element_type=jnp.float32)
         + jnp.dot(x_ref[1], wr_ref[DH:D, :],
                   preferred_element_type=jnp.float32))
    if relu:
      y = jnp.maximum(y, 0.0)
    if split_out:
      o_refs[0][0] = y[:, 0:DH]
      o_refs[0][1] = y[:, DH:D]
    else:
      o_refs[0][...] = y

  if split_out:
    out_shape = jax.ShapeDtypeStruct((NC, N, DH), jnp.float32)
    out_specs = pl.BlockSpec((NC, _R, DH), lambda i: (0, i, 0))
  else:
    out_shape = jax.ShapeDtypeStruct((N, D), jnp.float32)
    out_specs = pl.BlockSpec((_R, D), lambda i: (i, 0))

  return pl.pallas_call(
      body,
      grid=(N // _R,),
      in_specs=[
          pl.BlockSpec((NC, _R, DH), lambda i: (0, i, 0)),
          pl.BlockSpec((NC, _R, 16), lambda i: (0, i, 0)),
          pl.BlockSpec((NC, _R, DH), lambda i: (0, i, 0)),
          pl.BlockSpec((D, D), lambda i: (0, 0)),
          pl.BlockSpec((1, D), lambda i: (0, 0)),
          pl.BlockSpec((D, D), lambda i: (0, 0)),
      ],
      out_specs=out_specs,
      out_shape=out_shape,
  )


_dense_relu_split = _make_dense(True, True)
_dense_lin = _make_dense(False, False)


def kernel(x, edge_index, W1_l, b1_l, W1_r, W2_l, b2_l, W2_r):
  edges4 = edge_index.astype(jnp.int32).reshape(2, NS, NG, NBUF, CHUNK)
  xs = jnp.moveaxis(x.reshape(N, NC, DH), 1, 0)
  zrow = jnp.zeros((RPS, DH), jnp.float32)
  zcnt = jnp.zeros((RPS, 16), jnp.float32)
  ones = jnp.ones((CHUNK, 16), jnp.float32)
  b1 = b1_l.reshape(1, D)
  b2 = b2_l.reshape(1, D)

  agg1, cnt = _agg_counts(xs, edges4, zrow, zcnt, ones)
  hs = _dense_relu_split(agg1, cnt, xs, W1_l, b1, W1_r)
  agg2, = _agg_plain(hs, edges4, zrow)
  return _dense_lin(agg2, cnt, hs, W2_l, b2, W2_r)
